# Initial kernel scaffold; baseline (speedup 1.0000x reference)
import jax
import jax.numpy as jnp
from jax.experimental import pallas as pl


def kernel(z_int, norm, codebook):
    raise NotImplementedError("write your pallas kernel here")


# speedup vs baseline: 1.7978x; 1.7978x over previous
import functools

import jax
import jax.numpy as jnp
from jax import lax
from jax.experimental import pallas as pl
from jax.experimental.pallas import tpu as pltpu
from jax.experimental.pallas import tpu_sc as plsc

H = 30.0
BQ = 256
SC_WORKERS = 32


def _tc_body(z_ref, cabs_ref, sg_ref, norm_ref, dist_ref, idx_ref):
    tp = pl.program_id(1)
    ntp = pl.num_programs(1)

    z = z_ref[...]
    lane = lax.broadcasted_iota(jnp.int32, z.shape, 1)
    zcol = jnp.sum(jnp.where(lane == tp, z, 0), axis=1, keepdims=True)

    cabs = cabs_ref[...]
    sg = sg_ref[...]
    row = lax.broadcasted_iota(jnp.int32, cabs.shape, 0)
    crow = jnp.sum(jnp.where(row == tp, cabs, 0), axis=0, keepdims=True)
    sgrow = jnp.sum(jnp.where(row == tp, sg, 0.0), axis=0, keepdims=True)

    x = jnp.bitwise_xor(zcol, crow)
    f = (x + 1).astype(jnp.float32)
    bits = lax.bitcast_convert_type(f, jnp.int32)
    e = lax.shift_right_logical(bits, 23) - 126
    s = e.astype(jnp.float32) / H
    w = 1.0 - s
    t = (sgrow * w) * norm_ref[...]

    @pl.when(tp == 0)
    def _():
        dist_ref[...] = jnp.zeros_like(dist_ref)

    dist_ref[...] += t

    @pl.when(tp == ntp - 1)
    def _():
        d = dist_ref[...]
        m = jnp.min(d, axis=1, keepdims=True)
        kio = lax.broadcasted_iota(jnp.int32, d.shape, 1)
        idx_ref[...] = jnp.min(
            jnp.where(d == m, kio, d.shape[1]), axis=1, keepdims=True
        )


def _tc_call(z_int, cabs_t, sg_t, norm):
    q, tpd = z_int.shape
    k = norm.shape[1]
    dist, idx2 = pl.pallas_call(
        _tc_body,
        grid=(q // BQ, tpd),
        in_specs=[
            pl.BlockSpec((BQ, tpd), lambda i, t: (i, 0)),
            pl.BlockSpec((tpd, k), lambda i, t: (0, 0)),
            pl.BlockSpec((tpd, k), lambda i, t: (0, 0)),
            pl.BlockSpec((BQ, k), lambda i, t: (i, 0)),
        ],
        out_specs=[
            pl.BlockSpec((BQ, k), lambda i, t: (i, 0)),
            pl.BlockSpec((BQ, 1), lambda i, t: (i, 0)),
        ],
        out_shape=[
            jax.ShapeDtypeStruct((q, k), jnp.float32),
            jax.ShapeDtypeStruct((q, 1), jnp.int32),
        ],
        compiler_params=pltpu.CompilerParams(
            dimension_semantics=("parallel", "arbitrary"),
        ),
    )(z_int, cabs_t, sg_t, norm)
    return dist, idx2[:, 0]


def _sc_gather(codebook, idx):
    q = idx.shape[0]
    k, tpd = codebook.shape
    bpw = q // SC_WORKERS
    mesh = plsc.VectorSubcoreMesh(core_axis_name="c", subcore_axis_name="s")

    @functools.partial(
        pl.kernel,
        mesh=mesh,
        out_type=jax.ShapeDtypeStruct((q, tpd), jnp.int32),
        compiler_params=pltpu.CompilerParams(use_tc_tiling_on_sc=False),
        scratch_types=[
            pltpu.VMEM((bpw,), jnp.int32),
            pltpu.VMEM((bpw, tpd), jnp.int32),
            pltpu.SemaphoreType.DMA,
        ],
    )
    def gk(cb_hbm, idx_hbm, out_hbm, idx_v, rows_v, sem):
        wid = lax.axis_index("s") * 2 + lax.axis_index("c")
        base = wid * bpw
        pltpu.sync_copy(idx_hbm.at[pl.ds(base, bpw)], idx_v)
        pltpu.async_copy(cb_hbm.at[idx_v], rows_v, sem).wait()
        pltpu.sync_copy(rows_v, out_hbm.at[pl.ds(base, bpw)])

    return gk(codebook, idx)


def kernel(z_int, norm, codebook):
    cabs_t = jnp.abs(codebook).T
    sg_t = (2 * (codebook >= 0).astype(jnp.int32) - 1).astype(jnp.float32).T
    dist, idx = _tc_call(z_int, cabs_t, sg_t, norm)
    quant = _sc_gather(codebook, idx)
    return dist, quant, idx


# speedup vs baseline: 2.1239x; 1.1814x over previous
import functools

import jax
import jax.numpy as jnp
from jax import lax
from jax.experimental import pallas as pl
from jax.experimental.pallas import tpu as pltpu
from jax.experimental.pallas import tpu_sc as plsc

H = 30.0
BQ = 128
NS = 8
NV = 4
SC_WORKERS = 32


def _tc_body(zg_ref, cab_ref, sgn_ref, norm_ref, dist_ref, idx_ref, pacc_ref):
    s = pl.program_id(1)

    z4 = zg_ref[0]
    cab4 = cab_ref[0]
    sgn4 = sgn_ref[0]
    nrm = norm_ref[...]

    acc = None
    for v in range(NV):
        zcol = z4[:, v : v + 1]
        crow = cab4[v : v + 1, :]
        srow = sgn4[v : v + 1, :]
        x = jnp.bitwise_xor(zcol, crow)
        f = (x + 1).astype(jnp.float32)
        bits = lax.bitcast_convert_type(f, jnp.int32)
        e = lax.shift_right_logical(bits, 23) - 126
        w = 1.0 - e.astype(jnp.float32) / H
        wn = w * nrm
        t = lax.bitcast_convert_type(
            jnp.bitwise_xor(lax.bitcast_convert_type(wn, jnp.int32), srow),
            jnp.float32,
        )
        acc = t if acc is None else acc + t

    @pl.when(s < NS - 1)
    def _():
        pacc_ref[pl.ds(s, 1)] = acc[None]

    @pl.when(s == NS - 1)
    def _():
        p = [pacc_ref[j] for j in range(NS - 1)] + [acc]
        b = [p[j] + p[j + 4] for j in range(4)]
        c = [b[j] + b[j + 2] for j in range(2)]
        d = c[0] + c[1]
        dist_ref[...] = d
        m = jnp.min(d, axis=1, keepdims=True)
        kio = lax.broadcasted_iota(jnp.int32, d.shape, 1)
        idx_ref[...] = jnp.min(
            jnp.where(d == m, kio, d.shape[1]), axis=1, keepdims=True
        )


def _tc_call(z_g, cab_g, sgn_g, norm):
    q, k = norm.shape
    dist, idx2 = pl.pallas_call(
        _tc_body,
        grid=(q // BQ, NS),
        in_specs=[
            pl.BlockSpec((1, BQ, NV), lambda i, s: (s, i, 0)),
            pl.BlockSpec((1, NV, k), lambda i, s: (s, 0, 0)),
            pl.BlockSpec((1, NV, k), lambda i, s: (s, 0, 0)),
            pl.BlockSpec((BQ, k), lambda i, s: (i, 0)),
        ],
        out_specs=[
            pl.BlockSpec((BQ, k), lambda i, s: (i, 0)),
            pl.BlockSpec((BQ, 1), lambda i, s: (i, 0)),
        ],
        out_shape=[
            jax.ShapeDtypeStruct((q, k), jnp.float32),
            jax.ShapeDtypeStruct((q, 1), jnp.int32),
        ],
        scratch_shapes=[pltpu.VMEM((NS - 1, BQ, k), jnp.float32)],
        compiler_params=pltpu.CompilerParams(
            dimension_semantics=("parallel", "arbitrary"),
        ),
    )(z_g, cab_g, sgn_g, norm)
    return dist, idx2[:, 0]


def _sc_gather(codebook, idx):
    q = idx.shape[0]
    k, tpd = codebook.shape
    bpw = q // SC_WORKERS
    mesh = plsc.VectorSubcoreMesh(core_axis_name="c", subcore_axis_name="s")

    @functools.partial(
        pl.kernel,
        mesh=mesh,
        out_type=jax.ShapeDtypeStruct((q, tpd), jnp.int32),
        compiler_params=pltpu.CompilerParams(use_tc_tiling_on_sc=False),
        scratch_types=[
            pltpu.VMEM((bpw,), jnp.int32),
            pltpu.VMEM((bpw, tpd), jnp.int32),
            pltpu.SemaphoreType.DMA,
        ],
    )
    def gk(cb_hbm, idx_hbm, out_hbm, idx_v, rows_v, sem):
        wid = lax.axis_index("s") * 2 + lax.axis_index("c")
        base = wid * bpw
        pltpu.sync_copy(idx_hbm.at[pl.ds(base, bpw)], idx_v)
        pltpu.async_copy(cb_hbm.at[idx_v], rows_v, sem).wait()
        pltpu.sync_copy(rows_v, out_hbm.at[pl.ds(base, bpw)])

    return gk(codebook, idx)


def kernel(z_int, norm, codebook):
    q, tpd = z_int.shape
    k = codebook.shape[0]
    z_g = z_int.T.reshape(NV, NS, q).transpose(1, 2, 0)
    cabs_t = jnp.abs(codebook).T
    cab_g = cabs_t.reshape(NV, NS, k).transpose(1, 0, 2)
    sgn_t = jnp.where(codebook < 0, jnp.int32(-2147483648), 0).T
    sgn_g = sgn_t.reshape(NV, NS, k).transpose(1, 0, 2)
    dist, idx = _tc_call(z_g, cab_g, sgn_g, norm)
    quant = _sc_gather(codebook, idx)
    return dist, quant, idx


# speedup vs baseline: 2.3491x; 1.1060x over previous
import functools

import jax
import jax.numpy as jnp
from jax import lax
from jax.experimental import pallas as pl
from jax.experimental.pallas import tpu as pltpu
from jax.experimental.pallas import tpu_sc as plsc

H = 30.0
BQ = 512
NS = 8
NV = 4
SC_WORKERS = 32


def _tc_body(zg_ref, cab_ref, sgn_ref, norm_ref, dist_ref, idx_ref, pacc_ref):
    s = pl.program_id(1)

    z4 = zg_ref[0]
    cab4 = cab_ref[0]
    sgn4 = sgn_ref[0]
    nrm = norm_ref[...]

    acc = None
    for v in range(NV):
        zcol = z4[:, v : v + 1]
        crow = cab4[v : v + 1, :]
        srow = sgn4[v : v + 1, :]
        x = jnp.bitwise_xor(zcol, crow)
        f = (x + 1).astype(jnp.float32)
        bits = lax.bitcast_convert_type(f, jnp.int32)
        e = lax.shift_right_logical(bits, 23) - 126
        w = 1.0 - e.astype(jnp.float32) / H
        wn = w * nrm
        t = lax.bitcast_convert_type(
            jnp.bitwise_xor(lax.bitcast_convert_type(wn, jnp.int32), srow),
            jnp.float32,
        )
        acc = t if acc is None else acc + t

    @pl.when(s < NS - 1)
    def _():
        pacc_ref[pl.ds(s, 1)] = acc[None]

    @pl.when(s == NS - 1)
    def _():
        p = [pacc_ref[j] for j in range(NS - 1)] + [acc]
        b = [p[j] + p[j + 4] for j in range(4)]
        c = [b[j] + b[j + 2] for j in range(2)]
        d = c[0] + c[1]
        dist_ref[...] = d
        m = jnp.min(d, axis=1, keepdims=True)
        kio = lax.broadcasted_iota(jnp.int32, d.shape, 1)
        idx_ref[...] = jnp.min(
            jnp.where(d == m, kio, d.shape[1]), axis=1, keepdims=True
        )


def _tc_call(z_g, cab_g, sgn_g, norm):
    q, k = norm.shape
    dist, idx2 = pl.pallas_call(
        _tc_body,
        grid=(q // BQ, NS),
        in_specs=[
            pl.BlockSpec((1, BQ, NV), lambda i, s: (s, i, 0)),
            pl.BlockSpec((1, NV, k), lambda i, s: (s, 0, 0)),
            pl.BlockSpec((1, NV, k), lambda i, s: (s, 0, 0)),
            pl.BlockSpec((BQ, k), lambda i, s: (i, 0)),
        ],
        out_specs=[
            pl.BlockSpec((BQ, k), lambda i, s: (i, 0)),
            pl.BlockSpec((BQ, 1), lambda i, s: (i, 0)),
        ],
        out_shape=[
            jax.ShapeDtypeStruct((q, k), jnp.float32),
            jax.ShapeDtypeStruct((q, 1), jnp.int32),
        ],
        scratch_shapes=[pltpu.VMEM((NS - 1, BQ, k), jnp.float32)],
        compiler_params=pltpu.CompilerParams(
            dimension_semantics=("parallel", "arbitrary"),
        ),
    )(z_g, cab_g, sgn_g, norm)
    return dist, idx2[:, 0]


def _sc_gather(codebook, idx):
    q = idx.shape[0]
    k, tpd = codebook.shape
    bpw = q // SC_WORKERS
    mesh = plsc.VectorSubcoreMesh(core_axis_name="c", subcore_axis_name="s")

    @functools.partial(
        pl.kernel,
        mesh=mesh,
        out_type=jax.ShapeDtypeStruct((q, tpd), jnp.int32),
        compiler_params=pltpu.CompilerParams(use_tc_tiling_on_sc=False),
        scratch_types=[
            pltpu.VMEM((bpw,), jnp.int32),
            pltpu.VMEM((bpw, tpd), jnp.int32),
            pltpu.SemaphoreType.DMA,
        ],
    )
    def gk(cb_hbm, idx_hbm, out_hbm, idx_v, rows_v, sem):
        wid = lax.axis_index("s") * 2 + lax.axis_index("c")
        base = wid * bpw
        pltpu.sync_copy(idx_hbm.at[pl.ds(base, bpw)], idx_v)
        pltpu.async_copy(cb_hbm.at[idx_v], rows_v, sem).wait()
        pltpu.sync_copy(rows_v, out_hbm.at[pl.ds(base, bpw)])

    return gk(codebook, idx)


def kernel(z_int, norm, codebook):
    q, tpd = z_int.shape
    k = codebook.shape[0]
    z_g = z_int.T.reshape(NV, NS, q).transpose(1, 2, 0)
    cabs_t = jnp.abs(codebook).T
    cab_g = cabs_t.reshape(NV, NS, k).transpose(1, 0, 2)
    sgn_t = jnp.where(codebook < 0, jnp.int32(-2147483648), 0).T
    sgn_g = sgn_t.reshape(NV, NS, k).transpose(1, 0, 2)
    dist, idx = _tc_call(z_g, cab_g, sgn_g, norm)
    quant = _sc_gather(codebook, idx)
    return dist, quant, idx


# speedup vs baseline: 2.5602x; 1.0899x over previous
import functools

import jax
import jax.numpy as jnp
from jax import lax
from jax.experimental import pallas as pl
from jax.experimental.pallas import tpu as pltpu
from jax.experimental.pallas import tpu_sc as plsc

H = 30.0
BQ = 256
NS = 8
NV = 4
SGRID = 1
GPS = NS // SGRID
SC_WORKERS = 32


def _term(zcol, crow, srow, nrm):
    x = jnp.bitwise_xor(zcol, crow)
    f = (x + 1).astype(jnp.float32)
    bits = lax.bitcast_convert_type(f, jnp.int32)
    e = lax.shift_right_logical(bits, 23) - 126
    w = 1.0 - e.astype(jnp.float32) / H
    wn = w * nrm
    return lax.bitcast_convert_type(
        jnp.bitwise_xor(lax.bitcast_convert_type(wn, jnp.int32), srow),
        jnp.float32,
    )


def _combine(parts):
    n = len(parts)
    while n > 1:
        parts = [parts[j] + parts[j + n // 2] for j in range(n // 2)]
        n //= 2
    return parts[0]


def _tc_body(zg_ref, cab_ref, sgn_ref, norm_ref, dist_ref, idx_ref, pacc_ref):
    s = pl.program_id(1)

    zg = zg_ref[0]
    cab = cab_ref[0]
    sgn = sgn_ref[0]
    nrm = norm_ref[...]

    accs = []
    for h in range(GPS):
        acc = None
        for v in range(NV):
            j = h * NV + v
            t = _term(
                zg[:, j : j + 1], cab[j : j + 1, :], sgn[j : j + 1, :], nrm
            )
            acc = t if acc is None else acc + t
        accs.append(acc)
    part = _combine(accs)

    @pl.when(s < SGRID - 1)
    def _():
        pacc_ref[pl.ds(s, 1)] = part[None]

    @pl.when(s == SGRID - 1)
    def _():
        parts = [pacc_ref[j] for j in range(SGRID - 1)] + [part]
        d = _combine(parts)
        dist_ref[...] = d
        m = jnp.min(d, axis=1, keepdims=True)
        kio = lax.broadcasted_iota(jnp.int32, d.shape, 1)
        idx_ref[...] = jnp.min(
            jnp.where(d == m, kio, d.shape[1]), axis=1, keepdims=True
        )


def _tc_call(z_g, cab_g, sgn_g, norm):
    q, k = norm.shape
    ncols = GPS * NV
    dist, idx2 = pl.pallas_call(
        _tc_body,
        grid=(q // BQ, SGRID),
        in_specs=[
            pl.BlockSpec((1, BQ, ncols), lambda i, s: (s, i, 0)),
            pl.BlockSpec((1, ncols, k), lambda i, s: (s, 0, 0)),
            pl.BlockSpec((1, ncols, k), lambda i, s: (s, 0, 0)),
            pl.BlockSpec((BQ, k), lambda i, s: (i, 0)),
        ],
        out_specs=[
            pl.BlockSpec((BQ, k), lambda i, s: (i, 0)),
            pl.BlockSpec((BQ, 1), lambda i, s: (i, 0)),
        ],
        out_shape=[
            jax.ShapeDtypeStruct((q, k), jnp.float32),
            jax.ShapeDtypeStruct((q, 1), jnp.int32),
        ],
        scratch_shapes=[pltpu.VMEM((max(SGRID - 1, 1), BQ, k), jnp.float32)],
        compiler_params=pltpu.CompilerParams(
            dimension_semantics=("parallel", "arbitrary"),
        ),
    )(z_g, cab_g, sgn_g, norm)
    return dist, idx2[:, 0]


def _sc_gather(codebook, idx):
    q = idx.shape[0]
    k, tpd = codebook.shape
    bpw = q // SC_WORKERS
    mesh = plsc.VectorSubcoreMesh(core_axis_name="c", subcore_axis_name="s")

    @functools.partial(
        pl.kernel,
        mesh=mesh,
        out_type=jax.ShapeDtypeStruct((q, tpd), jnp.int32),
        compiler_params=pltpu.CompilerParams(use_tc_tiling_on_sc=False),
        scratch_types=[
            pltpu.VMEM((bpw,), jnp.int32),
            pltpu.VMEM((bpw, tpd), jnp.int32),
            pltpu.SemaphoreType.DMA,
        ],
    )
    def gk(cb_hbm, idx_hbm, out_hbm, idx_v, rows_v, sem):
        wid = lax.axis_index("s") * 2 + lax.axis_index("c")
        base = wid * bpw
        pltpu.sync_copy(idx_hbm.at[pl.ds(base, bpw)], idx_v)
        pltpu.async_copy(cb_hbm.at[idx_v], rows_v, sem).wait()
        pltpu.sync_copy(rows_v, out_hbm.at[pl.ds(base, bpw)])

    return gk(codebook, idx)


def _regroup(rows, last_dim_minor):
    tp, n = rows.shape
    r = rows.reshape(NV, NS, n)
    r = r.reshape(NV, GPS, SGRID, n)
    r = r.transpose(2, 1, 0, 3).reshape(SGRID, GPS * NV, n)
    if last_dim_minor:
        return r
    return r.transpose(0, 2, 1)


def kernel(z_int, norm, codebook):
    z_g = _regroup(z_int.T, last_dim_minor=False)
    cab_g = _regroup(jnp.abs(codebook).T, last_dim_minor=True)
    sgn_t = jnp.where(codebook < 0, jnp.int32(-2147483648), 0).T
    sgn_g = _regroup(sgn_t, last_dim_minor=True)
    dist, idx = _tc_call(z_g, cab_g, sgn_g, norm)
    quant = _sc_gather(codebook, idx)
    return dist, quant, idx


# speedup vs baseline: 2.5833x; 1.0090x over previous
import functools

import jax
import jax.numpy as jnp
from jax import lax
from jax.experimental import pallas as pl
from jax.experimental.pallas import tpu as pltpu
from jax.experimental.pallas import tpu_sc as plsc

H = 30.0
BQ = 256
NS = 8
NV = 4
SC_WORKERS = 32


def _term(zcol, crow, srow, nrm):
    x = jnp.bitwise_xor(zcol, crow)
    f = (x + 1).astype(jnp.float32)
    bits = lax.bitcast_convert_type(f, jnp.int32)
    e = lax.shift_right_logical(bits, 23) - 126
    w = 1.0 - e.astype(jnp.float32) / H
    wn = w * nrm
    return lax.bitcast_convert_type(
        jnp.bitwise_xor(lax.bitcast_convert_type(wn, jnp.int32), srow),
        jnp.float32,
    )


def _combine(parts):
    n = len(parts)
    while n > 1:
        parts = [parts[j] + parts[j + n // 2] for j in range(n // 2)]
        n //= 2
    return parts[0]


def _tc_body(z_ref, cab_ref, sgn_ref, norm_ref, dist_ref, idx_ref):
    z = z_ref[...]
    cab = cab_ref[...]
    sgn = sgn_ref[...]
    nrm = norm_ref[...]

    accs = []
    for g in range(NS):
        acc = None
        for v in range(NV):
            tp = v * NS + g
            t = _term(
                z[:, tp : tp + 1], cab[tp : tp + 1, :], sgn[tp : tp + 1, :], nrm
            )
            acc = t if acc is None else acc + t
        accs.append(acc)
    d = _combine(accs)
    dist_ref[...] = d
    m = jnp.min(d, axis=1, keepdims=True)
    kio = lax.broadcasted_iota(jnp.int32, d.shape, 1)
    idx_ref[...] = jnp.min(
        jnp.where(d == m, kio, d.shape[1]), axis=1, keepdims=True
    )


def _tc_call(z_int, cab, sgn, norm):
    q, k = norm.shape
    tp = z_int.shape[1]
    dist, idx2 = pl.pallas_call(
        _tc_body,
        grid=(q // BQ,),
        in_specs=[
            pl.BlockSpec((BQ, tp), lambda i: (i, 0)),
            pl.BlockSpec((tp, k), lambda i: (0, 0)),
            pl.BlockSpec((tp, k), lambda i: (0, 0)),
            pl.BlockSpec((BQ, k), lambda i: (i, 0)),
        ],
        out_specs=[
            pl.BlockSpec((BQ, k), lambda i: (i, 0)),
            pl.BlockSpec((BQ, 1), lambda i: (i, 0)),
        ],
        out_shape=[
            jax.ShapeDtypeStruct((q, k), jnp.float32),
            jax.ShapeDtypeStruct((q, 1), jnp.int32),
        ],
        compiler_params=pltpu.CompilerParams(
            dimension_semantics=("parallel",),
        ),
    )(z_int, cab, sgn, norm)
    return dist, idx2[:, 0]


def _sc_gather(codebook, idx):
    q = idx.shape[0]
    k, tpd = codebook.shape
    bpw = q // SC_WORKERS
    mesh = plsc.VectorSubcoreMesh(core_axis_name="c", subcore_axis_name="s")

    @functools.partial(
        pl.kernel,
        mesh=mesh,
        out_type=jax.ShapeDtypeStruct((q, tpd), jnp.int32),
        compiler_params=pltpu.CompilerParams(use_tc_tiling_on_sc=False),
        scratch_types=[
            pltpu.VMEM((bpw,), jnp.int32),
            pltpu.VMEM((bpw, tpd), jnp.int32),
            pltpu.SemaphoreType.DMA,
        ],
    )
    def gk(cb_hbm, idx_hbm, out_hbm, idx_v, rows_v, sem):
        wid = lax.axis_index("s") * 2 + lax.axis_index("c")
        base = wid * bpw
        pltpu.sync_copy(idx_hbm.at[pl.ds(base, bpw)], idx_v)
        pltpu.async_copy(cb_hbm.at[idx_v], rows_v, sem).wait()
        pltpu.sync_copy(rows_v, out_hbm.at[pl.ds(base, bpw)])

    return gk(codebook, idx)


def kernel(z_int, norm, codebook):
    cab = jnp.abs(codebook).T
    sgn = jnp.where(codebook < 0, jnp.int32(-2147483648), 0).T
    dist, idx = _tc_call(z_int, cab, sgn, norm)
    quant = _sc_gather(codebook, idx)
    return dist, quant, idx
